# baseline (speedup 1.0000x reference)
import functools

import jax
import jax.numpy as jnp
from jax import lax
from jax.experimental import pallas as pl
from jax.experimental.pallas import tpu as pltpu
from jax.experimental.pallas import tpu_sc as plsc

_B = 4096
_L = 200
_V = 100000
_D = 64

_NC = 2
_NS = 16
_NW = _NC * _NS
_CL = 2
_NCH = _L // _CL
_RPC = _CL * 128

_mesh = plsc.VectorSubcoreMesh(core_axis_name="c", subcore_axis_name="s")


@functools.partial(
    pl.kernel,
    mesh=_mesh,
    out_type=jax.ShapeDtypeStruct((_L * 8, _NW, 8, 128), jnp.float32),
    scratch_types=[
        pltpu.VMEM((_CL, 128), jnp.int32),
        pltpu.VMEM((_CL, 128), jnp.int32),
        pltpu.VMEM((_RPC, _D), jnp.float32),
        pltpu.VMEM((_RPC, _D), jnp.float32),
        pltpu.VMEM((_CL * 8, 8, 128), jnp.float32),
        pltpu.VMEM((_CL * 8, 8, 128), jnp.float32),
        pltpu.VMEM((_L, _D), jnp.float32),
        pltpu.SemaphoreType.DMA,
        pltpu.SemaphoreType.DMA,
        pltpu.SemaphoreType.DMA,
        pltpu.SemaphoreType.DMA,
    ],
    compiler_params=pltpu.CompilerParams(use_tc_tiling_on_sc=False, needs_layout_passes=False),
)
def _sc_encode(idx_hbm, table_hbm, pos_hbm, out_hbm,
               idx_v0, idx_v1, rows_v0, rows_v1, tr_v0, tr_v1, pos_v,
               gsem0, gsem1, ssem0, ssem1):
    w = lax.axis_index("s") * _NC + lax.axis_index("c")
    pltpu.sync_copy(pos_hbm, pos_v)
    lanes = lax.iota(jnp.int32, 16)
    bufs = (
        (idx_v0, rows_v0, tr_v0, gsem0, ssem0),
        (idx_v1, rows_v1, tr_v1, gsem1, ssem1),
    )

    def fire_gathers(c, idx_v, rows_v, gsem):
        pltpu.sync_copy(
            idx_hbm.at[pl.ds(c * _CL, _CL), pl.ds(w * 128, 128)], idx_v
        )
        for j in range(_CL):
            pltpu.async_copy(
                table_hbm.at[idx_v.at[j]],
                rows_v.at[pl.ds(j * 128, 128)],
                gsem,
            )

    def wait_gathers(rows_v, gsem):
        pltpu.make_async_copy(table_hbm.at[pl.ds(0, _RPC)], rows_v, gsem).wait()

    def wait_store(tr_v, ssem):
        pltpu.make_async_copy(out_hbm.at[pl.ds(0, _CL * 8), 0], tr_v, ssem).wait()

    def transpose_add(c, rows_v, tr_v):
        for lc in range(_CL):
            l = c * _CL + lc
            base = lc * 128
            lrow = jnp.full((16,), l, jnp.int32)

            def dgbody(dg, carry):
                for dl in range(8):
                    dcol = jnp.full((16,), dg * 8 + dl, jnp.int32)
                    pos16 = plsc.load_gather(pos_v, [lrow, dcol])
                    for v in range(8):
                        x = plsc.load_gather(
                            rows_v, [base + v * 16 + lanes, dcol]
                        )
                        tr_v[lc * 8 + dg, dl, pl.ds(v * 16, 16)] = x + pos16
                return carry

            lax.fori_loop(0, 8, dgbody, 0)

    def fire_store(c, tr_v, ssem):
        pltpu.async_copy(
            tr_v, out_hbm.at[pl.ds(c * _CL * 8, _CL * 8), w], ssem
        )

    fire_gathers(0, idx_v0, rows_v0, gsem0)

    def outer(g, carry):
        for b in range(2):
            c = 2 * g + b
            idx_p, rows_p, tr_p, gsem_p, ssem_p = bufs[b]
            idx_q, rows_q, tr_q, gsem_q, ssem_q = bufs[1 - b]

            @pl.when(c + 1 < _NCH)
            def _():
                fire_gathers(c + 1, idx_q, rows_q, gsem_q)

            wait_gathers(rows_p, gsem_p)

            @pl.when(c >= 2)
            def _():
                wait_store(tr_p, ssem_p)

            transpose_add(c, rows_p, tr_p)
            fire_store(c, tr_p, ssem_p)
        return carry

    lax.fori_loop(0, _NCH // 2, outer, 0)
    wait_store(tr_v0, ssem0)
    wait_store(tr_v1, ssem1)


def kernel(indices, table, pos_emb):
    idx_t = indices.T.astype(jnp.int32)
    out = _sc_encode(idx_t, table, pos_emb)
    out = out.reshape(_L, 8, _NW, 8, 128).transpose(2, 4, 0, 1, 3)
    return out.reshape(_B, _L, _D)


# speedup vs baseline: 3.6000x; 3.6000x over previous
import functools

import jax
import jax.numpy as jnp
from jax import lax
from jax.experimental import pallas as pl
from jax.experimental.pallas import tpu as pltpu
from jax.experimental.pallas import tpu_sc as plsc

_B = 4096
_L = 200
_V = 100000
_D = 64

_NC = 2
_NS = 16
_NW = _NC * _NS
_N = _B * _L
_RB = 4
_C = _RB * _L
_GRP = 8
_GSZ = _C // _GRP
_NCHUNKS = _N // _C
_CH_PER_W = _NCHUNKS // _NW

_mesh = plsc.VectorSubcoreMesh(core_axis_name="c", subcore_axis_name="s")


@functools.partial(
    pl.kernel,
    mesh=_mesh,
    out_type=jax.ShapeDtypeStruct((_N, 2 * _D), jnp.float32),
    scratch_types=[
        pltpu.VMEM((_GRP, _GSZ), jnp.int32),
        pltpu.VMEM((_GRP, _GSZ), jnp.int32),
        pltpu.VMEM((_C, _D), jnp.float32),
        pltpu.VMEM((_C, _D), jnp.float32),
        pltpu.VMEM((_L, _D), jnp.float32),
        pltpu.SemaphoreType.DMA,
        pltpu.SemaphoreType.DMA,
        pltpu.SemaphoreType.DMA,
        pltpu.SemaphoreType.DMA,
    ],
    compiler_params=pltpu.CompilerParams(use_tc_tiling_on_sc=False),
)
def _sc_encode(idx_hbm, table_hbm, pos_hbm, out_hbm,
               idx_v0, idx_v1, rows_v0, rows_v1, pos_v,
               gsem0, gsem1, ssem0, ssem1):
    wid = lax.axis_index("s") * _NC + lax.axis_index("c")
    pltpu.sync_copy(pos_hbm, pos_v)
    bufs = ((idx_v0, rows_v0, gsem0, ssem0), (idx_v1, rows_v1, gsem1, ssem1))

    def fire_gathers(c, idx_v, rows_v, gsem):
        chunk = wid * _CH_PER_W + c
        pltpu.sync_copy(idx_hbm.at[pl.ds(chunk * _GRP, _GRP)], idx_v)
        for j in range(_GRP):
            pltpu.async_copy(
                table_hbm.at[idx_v.at[j]],
                rows_v.at[pl.ds(j * _GSZ, _GSZ)],
                gsem,
            )

    def wait_gathers(rows_v, gsem):
        pltpu.make_async_copy(table_hbm.at[pl.ds(0, _C)], rows_v, gsem).wait()

    def wait_store(rows_v, ssem):
        pltpu.make_async_copy(table_hbm.at[pl.ds(0, _C)], rows_v, ssem).wait()

    def fire_store(c, rows_v, ssem):
        chunk = wid * _CH_PER_W + c
        pltpu.async_copy(
            rows_v,
            out_hbm.at[pl.ds(chunk * _C, _C), pl.ds(0, _D)],
            ssem,
        )

    def add_pos(rows_v):
        def pos_body(p, c2):
            pv = [pos_v[p, pl.ds(16 * k, 16)] for k in range(4)]
            for r in range(_RB):
                for k in range(4):
                    rows_v[r * _L + p, pl.ds(16 * k, 16)] += pv[k]
            return c2

        lax.fori_loop(0, _L, pos_body, 0)

    fire_gathers(0, idx_v0, rows_v0, gsem0)

    def outer(g, carry):
        for b in range(2):
            c = 2 * g + b
            idx_p, rows_p, gsem_p, ssem_p = bufs[b]
            idx_q, rows_q, gsem_q, ssem_q = bufs[1 - b]

            @pl.when(c + 1 < _CH_PER_W)
            def _():
                @pl.when(c >= 1)
                def _():
                    wait_store(rows_q, ssem_q)

                fire_gathers(c + 1, idx_q, rows_q, gsem_q)

            wait_gathers(rows_p, gsem_p)
            add_pos(rows_p)
            fire_store(c, rows_p, ssem_p)
        return carry

    lax.fori_loop(0, _CH_PER_W // 2, outer, 0)
    wait_store(rows_v0, ssem0)
    wait_store(rows_v1, ssem1)


def kernel(indices, table, pos_emb):
    idx2 = indices.reshape(_N // _GSZ, _GSZ).astype(jnp.int32)
    out = _sc_encode(idx2, table, pos_emb)
    return out.reshape(_B, _L, 2 * _D)[:, :, :_D]
